# Initial kernel scaffold; baseline (speedup 1.0000x reference)
import jax
import jax.numpy as jnp
from jax.experimental import pallas as pl


def kernel(noisy, clean, deno, fflow, bflow, curr_epoch):
    raise NotImplementedError("write your pallas kernel here")


# speedup vs baseline: 140.4257x; 140.4257x over previous
import functools

import jax
import jax.numpy as jnp
import numpy as np
from jax import lax
from jax.experimental import pallas as pl
from jax.experimental.pallas import tpu as pltpu
from jax.experimental.pallas import tpu_sc as plsc

_WS = 9
_PS = 3
_K = 16
_S0 = 2
_R = _WS // 2
_T, _F, _H, _W = 3, 3, 128, 128
_NH, _NW = _H // _S0, _W // _S0
_NOFF = _WS * _WS
_NPAD = 96
_NQ = _T * _NH * _NW
_NWORKERS = 32
_QB = _NQ // _NWORKERS
_BIG = 1e30


def _samp_mat():
    a = np.zeros((_NH, _H), np.float32)
    for i in range(_NH):
        for u in (2 * i - 1, 2 * i, 2 * i + 1):
            if 0 <= u < _H:
                a[i, u] = 1.0
    return a


def _dmap_body(p_ref, kp_ref, np_ref, a_ref, at_ref, dall_ref, dcross_ref):
    a = a_ref[...]
    at = at_ref[...]
    ps = [p_ref[0, c] for c in range(_F)]
    for o in range(_NOFF):
        dy, dx = o // _WS, o % _WS
        sqa = sqc = None
        for c in range(_F):
            ka = kp_ref[0, c, dy:dy + _H, dx:dx + _W]
            kn = np_ref[0, c, dy:dy + _H, dx:dx + _W]
            da = ps[c] - ka
            dn = ps[c] - kn
            sqa = da * da if sqa is None else sqa + da * da
            sqc = dn * dn if sqc is None else sqc + dn * dn
        dall_ref[0, o] = jnp.dot(
            jnp.dot(a, sqa, preferred_element_type=jnp.float32), at,
            preferred_element_type=jnp.float32)
        dcross_ref[0, o] = jnp.dot(
            jnp.dot(a, sqc, preferred_element_type=jnp.float32), at,
            preferred_element_type=jnp.float32)


def _dmaps(deno, noisy):
    kp = jnp.pad(deno, ((0, 0), (0, 0), (_R, _R), (_R, _R)), mode="edge")
    npd = jnp.pad(noisy, ((0, 0), (0, 0), (_R, _R), (_R, _R)), mode="edge")
    a = jnp.asarray(_samp_mat())
    at = a.T
    hp, wp = _H + 2 * _R, _W + 2 * _R
    return pl.pallas_call(
        _dmap_body,
        grid=(_T,),
        in_specs=[
            pl.BlockSpec((1, _F, _H, _W), lambda t: (t, 0, 0, 0)),
            pl.BlockSpec((1, _F, hp, wp), lambda t: (t, 0, 0, 0)),
            pl.BlockSpec((1, _F, hp, wp), lambda t: (t, 0, 0, 0)),
            pl.BlockSpec((_NH, _H), lambda t: (0, 0)),
            pl.BlockSpec((_H, _NH), lambda t: (0, 0)),
        ],
        out_specs=[
            pl.BlockSpec((1, _NOFF, _NH, _NW), lambda t: (t, 0, 0, 0)),
            pl.BlockSpec((1, _NOFF, _NH, _NW), lambda t: (t, 0, 0, 0)),
        ],
        out_shape=[
            jax.ShapeDtypeStruct((_T, _NOFF, _NH, _NW), jnp.float32),
            jax.ShapeDtypeStruct((_T, _NOFF, _NH, _NW), jnp.float32),
        ],
    )(deno, kp, npd, a, at)


def _topk_body(dall_hbm, dcross_hbm, out_hbm, dall_v, dcross_v, acc_v):
    nc = 2
    wid = lax.axis_index("s") * nc + lax.axis_index("c")
    base = wid * _QB
    pltpu.sync_copy(dall_hbm.at[pl.ds(base, _QB)], dall_v)
    pltpu.sync_copy(dcross_hbm.at[pl.ds(base, _QB)], dcross_v)

    nv = _NPAD // 16

    def body(q, acc):
        rk = rv = None
        for i in range(nv):
            k = dall_v[q, pl.ds(16 * i, 16)]
            v = dcross_v[q, pl.ds(16 * i, 16)]
            sk, sv = plsc.sort_key_val(k, v)
            if rk is None:
                rk, rv = sk, sv
            else:
                skr = lax.rev(sk, (0,))
                svr = lax.rev(sv, (0,))
                take = rk <= skr
                mk = jnp.where(take, rk, skr)
                mv = jnp.where(take, rv, svr)
                if i < nv - 1:
                    rk, rv = plsc.sort_key_val(mk, mv)
                else:
                    rk, rv = mk, mv
        msk = (rk / float(_PS * _PS * _F)) < 0.5
        return acc + jnp.where(msk, rv, jnp.float32(0.0))

    acc = lax.fori_loop(0, _QB, body, jnp.zeros((16,), jnp.float32))
    acc_v[...] = acc
    pltpu.sync_copy(acc_v, out_hbm.at[wid])


def _topk_partials(dall_q, dcross_q):
    mesh = plsc.VectorSubcoreMesh(core_axis_name="c", subcore_axis_name="s")
    fn = functools.partial(
        pl.kernel,
        out_type=jax.ShapeDtypeStruct((_NWORKERS, 16), jnp.float32),
        mesh=mesh,
        compiler_params=pltpu.CompilerParams(needs_layout_passes=False),
        scratch_types=[
            pltpu.VMEM((_QB, _NPAD), jnp.float32),
            pltpu.VMEM((_QB, _NPAD), jnp.float32),
            pltpu.VMEM((16,), jnp.float32),
        ],
    )(_topk_body)
    return fn(dall_q, dcross_q)


def kernel(noisy, clean, deno, fflow, bflow, curr_epoch):
    d = deno[0]
    n = noisy[0]
    dall, dcross = _dmaps(d, n)
    dall_q = dall.reshape(_T, _NOFF, _NH * _NW).transpose(0, 2, 1)
    dcross_q = dcross.reshape(_T, _NOFF, _NH * _NW).transpose(0, 2, 1)
    dall_q = dall_q.reshape(_NQ, _NOFF)
    dcross_q = dcross_q.reshape(_NQ, _NOFF)
    dall_q = jnp.pad(dall_q, ((0, 0), (0, _NPAD - _NOFF)),
                     constant_values=_BIG)
    dcross_q = jnp.pad(dcross_q, ((0, 0), (0, _NPAD - _NOFF)))
    parts = _topk_partials(dall_q, dcross_q)
    return jnp.sum(parts) / jnp.float32(_NQ * _K)
